# baseline (speedup 1.0000x reference)
import functools

import jax
import jax.numpy as jnp
from jax import lax
from jax.experimental import pallas as pl
from jax.experimental.pallas import tpu as pltpu
from jax.experimental.pallas import tpu_sc as plsc

_ROWS = 16384
_COLS = 2048

_SC_ROWS = 6144
_NC = 2
_NS = 16
_NW = _NC * _NS
_WROWS = _SC_ROWS // _NW
_CR = 16
_NCHUNK = _WROWS // _CR

_mesh = plsc.VectorSubcoreMesh(core_axis_name="c", subcore_axis_name="s")


@functools.partial(
    pl.kernel,
    out_type=jax.ShapeDtypeStruct((_SC_ROWS, _COLS), jnp.float32),
    mesh=_mesh,
    scratch_types=[
        pltpu.VMEM((_CR, _COLS), jnp.float32),
        pltpu.VMEM((_CR, _COLS), jnp.float32),
        pltpu.SemaphoreType.DMA((2,)),
        pltpu.SemaphoreType.DMA((2,)),
    ],
)
def _sc_copy(x_hbm, o_hbm, buf0, buf1, sem_in, sem_out):
    wid = lax.axis_index("s") * _NC + lax.axis_index("c")
    base = wid * _WROWS
    bufs = (buf0, buf1)

    def in_copy(j):
        b = j % 2
        sl = pl.ds(base + j * _CR, _CR)
        return pltpu.make_async_copy(x_hbm.at[sl], bufs[b], sem_in.at[b])

    def out_copy(j):
        b = j % 2
        sl = pl.ds(base + j * _CR, _CR)
        return pltpu.make_async_copy(bufs[b], o_hbm.at[sl], sem_out.at[b])

    in_copy(0).start()
    in_copy(1).start()
    for j in range(_NCHUNK):
        in_copy(j).wait()
        out_copy(j).start()
        if j + 2 < _NCHUNK:
            out_copy(j).wait()
            in_copy(j + 2).start()
    out_copy(_NCHUNK - 2).wait()
    out_copy(_NCHUNK - 1).wait()


_TC_ROWS = _ROWS - _SC_ROWS
_BLOCK_ROWS = 1024


def _tc_body(x_ref, o_ref):
    o_ref[...] = x_ref[...]


_SC_BLOCKS = _SC_ROWS // _BLOCK_ROWS


def _tc_copy(x2):
    return pl.pallas_call(
        _tc_body,
        grid=(_TC_ROWS // _BLOCK_ROWS,),
        in_specs=[pl.BlockSpec((_BLOCK_ROWS, _COLS), lambda i: (i + _SC_BLOCKS, 0))],
        out_specs=pl.BlockSpec((_BLOCK_ROWS, _COLS), lambda i: (i, 0)),
        out_shape=jax.ShapeDtypeStruct((_TC_ROWS, _COLS), x2.dtype),
        compiler_params=pltpu.CompilerParams(
            dimension_semantics=("arbitrary",),
        ),
    )(x2)


def kernel(x):
    x2 = x.reshape(_ROWS, _COLS)
    top = _sc_copy(x2)
    bot = _tc_copy(x2)
    return jnp.concatenate([top, bot], axis=0).reshape(x.shape)


# speedup vs baseline: 2.2103x; 2.2103x over previous
import jax
import jax.numpy as jnp
from jax.experimental import pallas as pl
from jax.experimental.pallas import tpu as pltpu

_ROWS = 16384
_COLS = 2048
_CR = 1024
_NBUF = 3
_NCHUNK = _ROWS // _CR

_tc_mesh = pltpu.create_tensorcore_mesh("tc")


@pl.kernel(
    out_type=jax.ShapeDtypeStruct((_ROWS, _COLS), jnp.float32),
    mesh=_tc_mesh,
    scratch_types=[
        [pltpu.VMEM((_CR, _COLS), jnp.float32) for _ in range(_NBUF)],
        pltpu.SemaphoreType.DMA((_NBUF,)),
        pltpu.SemaphoreType.DMA((_NBUF,)),
    ],
)
def _tc_copy(x_hbm, o_hbm, bufs, sem_in, sem_out):
    def in_copy(j):
        b = j % _NBUF
        sl = pl.ds(j * _CR, _CR)
        return pltpu.make_async_copy(x_hbm.at[sl], bufs[b], sem_in.at[b])

    def out_copy(j):
        b = j % _NBUF
        sl = pl.ds(j * _CR, _CR)
        return pltpu.make_async_copy(bufs[b], o_hbm.at[sl], sem_out.at[b])

    for j in range(_NBUF):
        in_copy(j).start()
    for j in range(_NCHUNK):
        in_copy(j).wait()
        out_copy(j).start()
        if j + _NBUF < _NCHUNK:
            out_copy(j).wait()
            in_copy(j + _NBUF).start()
    for j in range(_NCHUNK - _NBUF, _NCHUNK):
        out_copy(j).wait()


def kernel(x):
    out = _tc_copy(x.reshape(_ROWS, _COLS))
    return out.reshape(x.shape)


# speedup vs baseline: 2.2289x; 1.0084x over previous
import jax
import jax.numpy as jnp
from jax.experimental import pallas as pl
from jax.experimental.pallas import tpu as pltpu

_ROWS = 16384
_COLS = 2048
_BLOCK_ROWS = 1024


def _copy_body(x_ref, o_ref):
    o_ref[...] = x_ref[...]


def kernel(x):
    x2 = x.reshape(_ROWS, _COLS)
    out = pl.pallas_call(
        _copy_body,
        grid=(_ROWS // _BLOCK_ROWS,),
        in_specs=[pl.BlockSpec((_BLOCK_ROWS, _COLS), lambda i: (i, 0))],
        out_specs=pl.BlockSpec((_BLOCK_ROWS, _COLS), lambda i: (i, 0)),
        out_shape=jax.ShapeDtypeStruct((_ROWS, _COLS), x.dtype),
        compiler_params=pltpu.CompilerParams(
            dimension_semantics=("arbitrary",),
        ),
    )(x2)
    return out.reshape(x.shape)
